# Initial kernel scaffold; baseline (speedup 1.0000x reference)
import jax
import jax.numpy as jnp
from jax.experimental import pallas as pl


def kernel(hidden_states, params):
    raise NotImplementedError("write your pallas kernel here")


# speedup vs baseline: 1.6947x; 1.6947x over previous
import functools

import jax
import jax.numpy as jnp
from jax import lax
from jax.experimental import pallas as pl
from jax.experimental.pallas import tpu as pltpu
from jax.experimental.pallas import tpu_sc as plsc

B, S, H, HEADS, DH, FF, E, K = 2, 2048, 1024, 16, 64, 4096, 8, 2
T = B * S
EPS = 1e-12

BLK = 256
NB = 40
NPAD = NB * BLK
FFB = 2048
NFF = FF // FFB

NW = 32
GCH = 32


def _ln_rows(z, g, b):
    m = jnp.mean(z, axis=-1, keepdims=True)
    v = jnp.mean((z - m) ** 2, axis=-1, keepdims=True)
    return (z - m) / jnp.sqrt(v + EPS) * g + b


def _qkv_body(x_ref, w_ref, b_ref, o_ref):
    x = x_ref[...].astype(jnp.bfloat16)
    acc = jnp.dot(x, w_ref[...], preferred_element_type=jnp.float32)
    o_ref[...] = (acc + b_ref[...]).astype(jnp.bfloat16)


def _qkv(x2d, wqkv16, bqkv):
    return pl.pallas_call(
        _qkv_body,
        grid=(T // 256,),
        in_specs=[
            pl.BlockSpec((256, H), lambda i: (i, 0)),
            pl.BlockSpec((H, 3 * H), lambda i: (0, 0)),
            pl.BlockSpec((1, 3 * H), lambda i: (0, 0)),
        ],
        out_specs=pl.BlockSpec((256, 3 * H), lambda i: (i, 0)),
        out_shape=jax.ShapeDtypeStruct((T, 3 * H), jnp.bfloat16),
    )(x2d, wqkv16, bqkv)


def _attn_body(q_ref, k_ref, v_ref, o_ref):
    q = q_ref[0]
    k = k_ref[0]
    s = lax.dot_general(q, k, (((1,), (1,)), ((), ())),
                        preferred_element_type=jnp.float32) * 0.125
    m = jnp.max(s, axis=1, keepdims=True)
    p = jnp.exp(s - m)
    l = jnp.sum(p, axis=1, keepdims=True)
    a = (p / l).astype(jnp.bfloat16)
    ctx = jnp.dot(a, v_ref[0], preferred_element_type=jnp.float32)
    o_ref[0] = ctx.astype(jnp.bfloat16)


def _attention(qkvh):
    return pl.pallas_call(
        _attn_body,
        grid=(B, HEADS),
        in_specs=[
            pl.BlockSpec((1, S, DH), lambda b, h: (h, b, 0)),
            pl.BlockSpec((1, S, DH), lambda b, h: (h + HEADS, b, 0)),
            pl.BlockSpec((1, S, DH), lambda b, h: (h + 2 * HEADS, b, 0)),
        ],
        out_specs=pl.BlockSpec((1, S, DH), lambda b, h: (h, b, 0)),
        out_shape=jax.ShapeDtypeStruct((HEADS, T, DH), jnp.bfloat16),
    )(qkvh, qkvh, qkvh)


def _oproj_body(ctx_ref, x_ref, wo_ref, bo_ref, g_ref, b_ref, wg_ref,
                ao_ref, a16_ref, w1_ref, i1_ref, i2_ref):
    ctx = ctx_ref[...]
    z = jnp.dot(ctx, wo_ref[...], preferred_element_type=jnp.float32)
    z = z + bo_ref[...] + x_ref[...]
    a = _ln_rows(z, g_ref[...], b_ref[...])
    ao_ref[...] = a
    a16 = a.astype(jnp.bfloat16)
    a16_ref[...] = a16
    logits = jnp.dot(a16, wg_ref[...], preferred_element_type=jnp.float32)
    iota = lax.broadcasted_iota(jnp.int32, logits.shape, 1)
    m1 = jnp.max(logits, axis=1)
    sel1 = logits == m1[:, None]
    i1 = jnp.min(jnp.where(sel1, iota, E), axis=1)
    l2 = jnp.where(iota == i1[:, None], -1e30, logits)
    m2 = jnp.max(l2, axis=1)
    sel2 = l2 == m2[:, None]
    i2 = jnp.min(jnp.where(sel2, iota, E), axis=1)
    w1_ref[...] = 1.0 / (1.0 + jnp.exp(m2 - m1))
    i1_ref[...] = i1
    i2_ref[...] = i2


def _oproj_route(ctx, x2d, wo16, bo, ln_g, ln_b, wg):
    return pl.pallas_call(
        _oproj_body,
        grid=(T // 256,),
        in_specs=[
            pl.BlockSpec((256, H), lambda i: (i, 0)),
            pl.BlockSpec((256, H), lambda i: (i, 0)),
            pl.BlockSpec((H, H), lambda i: (0, 0)),
            pl.BlockSpec((1, H), lambda i: (0, 0)),
            pl.BlockSpec((1, H), lambda i: (0, 0)),
            pl.BlockSpec((1, H), lambda i: (0, 0)),
            pl.BlockSpec((H, E), lambda i: (0, 0)),
        ],
        out_specs=[
            pl.BlockSpec((256, H), lambda i: (i, 0)),
            pl.BlockSpec((256, H), lambda i: (i, 0)),
            pl.BlockSpec((256,), lambda i: (i,)),
            pl.BlockSpec((256,), lambda i: (i,)),
            pl.BlockSpec((256,), lambda i: (i,)),
        ],
        out_shape=[
            jax.ShapeDtypeStruct((T, H), jnp.float32),
            jax.ShapeDtypeStruct((T, H), jnp.bfloat16),
            jax.ShapeDtypeStruct((T,), jnp.float32),
            jax.ShapeDtypeStruct((T,), jnp.int32),
            jax.ShapeDtypeStruct((T,), jnp.int32),
        ],
    )(ctx, x2d, wo16, bo, ln_g, ln_b, wg)


def _sc_gather_rows(table, idx, n_out):
    W = table.shape[1]
    per_w = n_out // NW
    nch = per_w // GCH
    idx3 = idx.reshape(NW, nch, GCH)
    mesh = plsc.VectorSubcoreMesh(core_axis_name="c", subcore_axis_name="s")

    @functools.partial(
        pl.kernel,
        mesh=mesh,
        out_type=jax.ShapeDtypeStruct((n_out, W), table.dtype),
        scratch_types=[
            pltpu.VMEM((nch, GCH), jnp.int32),
            pltpu.VMEM((GCH, W), table.dtype),
            pltpu.VMEM((GCH, W), table.dtype),
            pltpu.SemaphoreType.DMA,
            pltpu.SemaphoreType.DMA,
        ],
    )
    def gather_kernel(table_hbm, idx_hbm, out_hbm, idx_v, buf0, buf1,
                      gsem0, gsem1):
        wid = lax.axis_index("s") * 2 + lax.axis_index("c")
        base = wid * per_w
        pltpu.sync_copy(idx_hbm.at[wid], idx_v)
        bufs = (buf0, buf1)
        sems = (gsem0, gsem1)
        cps = [None, None]
        cps[0] = pltpu.async_copy(table_hbm.at[idx_v.at[0]], buf0, gsem0)
        for c in range(nch):
            if c + 1 < nch:
                cps[(c + 1) % 2] = pltpu.async_copy(
                    table_hbm.at[idx_v.at[c + 1]], bufs[(c + 1) % 2],
                    sems[(c + 1) % 2])
            cps[c % 2].wait()
            pltpu.sync_copy(bufs[c % 2], out_hbm.at[pl.ds(base + c * GCH, GCH)])

    return gather_kernel(table, idx3)


def _moe_body(be_ref, x_ref, w1_ref, b1_ref, w2_ref, b2_ref, y_ref):
    j = pl.program_id(1)
    x = x_ref[...]
    h = jnp.dot(x, w1_ref[0], preferred_element_type=jnp.float32)
    h = h + b1_ref[0]
    h = (0.5 * h * (1.0 + lax.erf(h * 0.7071067811865476))).astype(
        jnp.bfloat16)
    yp = jnp.dot(h, w2_ref[0], preferred_element_type=jnp.float32)

    @pl.when(j == 0)
    def _():
        y_ref[...] = yp + b2_ref[0]

    @pl.when(j != 0)
    def _():
        y_ref[...] += yp


def _moe_ffn(x_sorted16, w1_16, b1, w2_16, b2, block_expert):
    grid_spec = pltpu.PrefetchScalarGridSpec(
        num_scalar_prefetch=1,
        grid=(NB, NFF),
        in_specs=[
            pl.BlockSpec((BLK, H), lambda i, j, be: (i, 0)),
            pl.BlockSpec((1, H, FFB), lambda i, j, be: (be[i], 0, j)),
            pl.BlockSpec((1, 1, FFB), lambda i, j, be: (be[i], 0, j)),
            pl.BlockSpec((1, FFB, H), lambda i, j, be: (be[i], j, 0)),
            pl.BlockSpec((1, 1, H), lambda i, j, be: (be[i], 0, 0)),
        ],
        out_specs=pl.BlockSpec((BLK, H), lambda i, j, be: (i, 0)),
    )
    return pl.pallas_call(
        _moe_body,
        grid_spec=grid_spec,
        out_shape=jax.ShapeDtypeStruct((NPAD, H), jnp.float32),
    )(block_expert, x_sorted16, w1_16, b1.reshape(E, 1, FF),
      w2_16, b2.reshape(E, 1, H))


def _combine_body(y0_ref, y1_ref, w1_ref, ao_ref, g_ref, b_ref, o_ref):
    w1 = w1_ref[...][:, None]
    moe = w1 * y0_ref[...] + (1.0 - w1) * y1_ref[...]
    o_ref[...] = _ln_rows(moe + ao_ref[...], g_ref[...], b_ref[...])


def _combine(y0, y1, w1, attn_out, ln2_g, ln2_b):
    return pl.pallas_call(
        _combine_body,
        grid=(T // 256,),
        in_specs=[
            pl.BlockSpec((256, H), lambda i: (i, 0)),
            pl.BlockSpec((256, H), lambda i: (i, 0)),
            pl.BlockSpec((256,), lambda i: (i,)),
            pl.BlockSpec((256, H), lambda i: (i, 0)),
            pl.BlockSpec((1, H), lambda i: (0, 0)),
            pl.BlockSpec((1, H), lambda i: (0, 0)),
        ],
        out_specs=pl.BlockSpec((256, H), lambda i: (i, 0)),
        out_shape=jax.ShapeDtypeStruct((T, H), jnp.float32),
    )(y0, y1, w1, attn_out, ln2_g, ln2_b)


def kernel(hidden_states, params):
    p = params
    x2d = hidden_states.reshape(T, H)

    wqkv16 = jnp.concatenate([p["Wq"], p["Wk"], p["Wv"]], axis=1).astype(
        jnp.bfloat16)
    bqkv = jnp.concatenate([p["bq"], p["bk"], p["bv"]])[None, :]
    wo16 = p["Wo"].astype(jnp.bfloat16)
    w1_16 = p["W1"].astype(jnp.bfloat16)
    w2_16 = p["W2"].astype(jnp.bfloat16)

    qkv = _qkv(x2d, wqkv16, bqkv)
    qkvh = qkv.reshape(T, 3 * HEADS, DH).transpose(1, 0, 2)
    ctxh = _attention(qkvh)
    ctx = ctxh.transpose(1, 0, 2).reshape(T, H)
    attn_out, attn16, w1, i1, i2 = _oproj_route(
        ctx, x2d, wo16, p["bo"][None, :], p["ln_attn_g"][None, :],
        p["ln_attn_b"][None, :], p["Wg"].astype(jnp.bfloat16))

    e_all = jnp.stack([i1, i2], axis=1).reshape(2 * T)
    onehot = (e_all[:, None] == jnp.arange(E)[None, :]).astype(jnp.int32)
    csum = jnp.cumsum(onehot, axis=0)
    counts = csum[-1]
    rank = jnp.take_along_axis(csum, e_all[:, None], axis=1)[:, 0] - 1
    padded = ((counts + BLK - 1) // BLK) * BLK
    offs = jnp.concatenate([jnp.zeros((1,), jnp.int32),
                            jnp.cumsum(padded).astype(jnp.int32)])
    dest = offs[e_all] + rank
    gidx = jnp.zeros((NPAD,), jnp.int32).at[dest].set(
        jnp.arange(2 * T, dtype=jnp.int32) // 2)
    block_expert = jnp.clip(
        jnp.searchsorted(offs, jnp.arange(NB, dtype=jnp.int32) * BLK,
                         side="right").astype(jnp.int32) - 1, 0, E - 1)

    attn16_words = lax.bitcast_convert_type(
        attn16.reshape(T, H // 2, 2), jnp.int32)
    xs_words = _sc_gather_rows(attn16_words, gidx, NPAD)
    x_sorted16 = lax.bitcast_convert_type(
        xs_words, jnp.bfloat16).reshape(NPAD, H)
    y = _moe_ffn(x_sorted16, w1_16, p["b1"], w2_16, p["b2"], block_expert)

    dest_k = dest.reshape(T, 2).T.reshape(2 * T)
    y_pairs = _sc_gather_rows(y, dest_k, 2 * T)
    y0 = y_pairs[:T]
    y1 = y_pairs[T:]

    out = _combine(y0, y1, w1, attn_out, p["ln2_g"][None, :],
                   p["ln2_b"][None, :])
    return out.reshape(B, S, H)
